# baseline (speedup 1.0000x reference)
import functools

import jax
import jax.numpy as jnp
from jax.experimental import pallas as pl
from jax.experimental.pallas import tpu as pltpu


def _cheb1_body(a_ref, x0t_ref, out_ref):
    out_ref[0] = jnp.dot(a_ref[0], x0t_ref[...],
                         preferred_element_type=jnp.float32)


def _cheb2_body(a_ref, x1t_ref, x0t_ref, p_ref, *, bm, nsup):
    r = pl.program_id(0)
    i = pl.program_id(1)
    x1t = x1t_ref[0]
    rows = x1t_ref[0, pl.ds(r * bm, bm), :]
    acc = rows + 2.0 * jnp.dot(a_ref[0], x1t,
                               preferred_element_type=jnp.float32)

    @pl.when(i == 0)
    def _():
        p_ref[...] = acc + (1.0 - nsup) * x0t_ref[...]

    @pl.when(i != 0)
    def _():
        p_ref[...] += acc


def _stats_body(z_ref, c_ref, *, bi, bj, n):
    j = pl.program_id(0)
    zj = z_ref[pl.ds(j * bj, bj), :]

    def body(k, carry):
        m, dsum = carry
        zi = z_ref[pl.ds(k * bi, bi), :]
        tile = jax.lax.dot_general(
            zi, zj, (((1,), (1,)), ((), ())),
            preferred_element_type=jnp.float32)
        tile = jnp.maximum(tile, 0.0)
        tm = jnp.max(tile, axis=0, keepdims=True)
        m_new = jnp.maximum(m, tm)
        dsum = dsum * jnp.exp(m - m_new) + jnp.sum(
            jnp.exp(tile - m_new), axis=0, keepdims=True)
        return m_new, dsum

    m0 = jnp.zeros((1, bj), jnp.float32)
    d0 = jnp.zeros((1, bj), jnp.float32)
    m, dsum = jax.lax.fori_loop(0, n // bi, body, (m0, d0))
    c_ref[...] = m + jnp.log(dsum)


def _final_body(z_ref, c_ref, x0t_ref, p_ref, w_ref, out_ref,
                *, bi, bj, n, batch, d):
    r = pl.program_id(0)
    zi = z_ref[pl.ds(r * bi, bi), :]
    bd = batch * d

    def body(k, acc):
        zj = z_ref[pl.ds(k * bj, bj), :]
        tile = jax.lax.dot_general(
            zi, zj, (((1,), (1,)), ((), ())),
            preferred_element_type=jnp.float32)
        tile = jnp.maximum(tile, 0.0)
        e = jnp.exp(tile - c_ref[:, pl.ds(k * bj, bj)])
        v = x0t_ref[pl.ds(k * bj, bj), :]
        return acc + jnp.dot(e, v, preferred_element_type=jnp.float32)

    xz = jax.lax.fori_loop(0, n // bj, body,
                           jnp.zeros((bi, bd), jnp.float32))
    s = xz + p_ref[...]
    w = w_ref[...]
    for b in range(batch):
        out_ref[b] = jnp.dot(s[:, b * d:(b + 1) * d], w,
                             preferred_element_type=jnp.float32)


def kernel(A, X, Z, W):
    nsup, n, _ = A.shape
    batch, d, _ = X.shape
    bd = batch * d
    out_f = W.shape[1]

    X0T = X.reshape(bd, n).T

    BM = 512
    BI = 512
    BJ = 512
    nb = n // BM

    x1t = pl.pallas_call(
        _cheb1_body,
        grid=(nsup, nb),
        in_specs=[
            pl.BlockSpec((1, BM, n), lambda i, r: (i, r, 0)),
            pl.BlockSpec((n, bd), lambda i, r: (0, 0)),
        ],
        out_specs=pl.BlockSpec((1, BM, bd), lambda i, r: (i, r, 0)),
        out_shape=jax.ShapeDtypeStruct((nsup, n, bd), jnp.float32),
        compiler_params=pltpu.CompilerParams(
            dimension_semantics=("arbitrary", "arbitrary")),
    )(A, X0T)

    p = pl.pallas_call(
        functools.partial(_cheb2_body, bm=BM, nsup=float(nsup)),
        grid=(nb, nsup),
        in_specs=[
            pl.BlockSpec((1, BM, n), lambda r, i: (i, r, 0)),
            pl.BlockSpec((1, n, bd), lambda r, i: (i, 0, 0)),
            pl.BlockSpec((BM, bd), lambda r, i: (r, 0)),
        ],
        out_specs=pl.BlockSpec((BM, bd), lambda r, i: (r, 0)),
        out_shape=jax.ShapeDtypeStruct((n, bd), jnp.float32),
        compiler_params=pltpu.CompilerParams(
            dimension_semantics=("arbitrary", "arbitrary")),
    )(A, x1t, X0T)

    c = pl.pallas_call(
        functools.partial(_stats_body, bi=BI, bj=BJ, n=n),
        grid=(n // BJ,),
        in_specs=[pl.BlockSpec((n, Z.shape[1]), lambda j: (0, 0))],
        out_specs=pl.BlockSpec((1, BJ), lambda j: (0, j)),
        out_shape=jax.ShapeDtypeStruct((1, n), jnp.float32),
    )(Z)

    out = pl.pallas_call(
        functools.partial(_final_body, bi=BI, bj=BJ, n=n, batch=batch, d=d),
        grid=(n // BI,),
        in_specs=[
            pl.BlockSpec((n, Z.shape[1]), lambda r: (0, 0)),
            pl.BlockSpec((1, n), lambda r: (0, 0)),
            pl.BlockSpec((n, bd), lambda r: (0, 0)),
            pl.BlockSpec((BI, bd), lambda r: (r, 0)),
            pl.BlockSpec((d, out_f), lambda r: (0, 0)),
        ],
        out_specs=pl.BlockSpec((batch, BI, out_f), lambda r: (0, r, 0)),
        out_shape=jax.ShapeDtypeStruct((batch, n, out_f), jnp.float32),
    )(Z, c, X0T, p, W)

    return out
